# baseline (speedup 1.0000x reference)
import functools

import jax
import jax.numpy as jnp
from jax import lax
from jax.experimental import pallas as pl
from jax.experimental.pallas import tpu as pltpu
from jax.experimental.pallas import tpu_sc as plsc

_NC = 2
_NS = 16
_NW = _NC * _NS


def _phi_all_body(a_ref, w_ref, b_ref, o_ref):
    o_ref[...] = (
        jnp.dot(a_ref[...], w_ref[...], preferred_element_type=jnp.float32)
        + b_ref[...]
    )


def _phi_all(atom, W, b, bn):
    N, D = atom.shape
    grid = N // bn
    return pl.pallas_call(
        _phi_all_body,
        grid=(grid,),
        in_specs=[
            pl.BlockSpec((bn, D), lambda i: (i, 0)),
            pl.BlockSpec((D, D), lambda i: (0, 0)),
            pl.BlockSpec((1, D), lambda i: (0, 0)),
        ],
        out_specs=pl.BlockSpec((bn, D), lambda i: (i, 0)),
        out_shape=jax.ShapeDtypeStruct((N, D), jnp.float32),
        compiler_params=pltpu.CompilerParams(
            dimension_semantics=("parallel",)
        ),
    )(atom, W, b)


def _sc_gather(table, idx_flat):
    N, D = table.shape
    B = idx_flat.shape[0]
    R = B // _NW
    C = 1
    for cand in (128, 120, 112, 104, 96, 88, 80, 72, 64, 56, 48, 40, 32,
                 24, 16, 8):
        if R % cand == 0:
            C = cand
            break
    K = R // C
    idx3 = idx_flat.reshape(_NW, K, C)

    mesh = plsc.VectorSubcoreMesh(
        core_axis_name="c", subcore_axis_name="s",
        num_cores=_NC, num_subcores=_NS,
    )

    @functools.partial(
        pl.kernel,
        mesh=mesh,
        out_type=jax.ShapeDtypeStruct((B, D), jnp.float32),
        scratch_types=[
            pltpu.VMEM((K, C), jnp.int32),
            pltpu.VMEM((2, C, D), jnp.float32),
            pltpu.SemaphoreType.DMA,
            pltpu.SemaphoreType.DMA,
        ],
    )
    def gather_k(table_hbm, idx_hbm, out_hbm, idx_v, rows_v, gsem, wsem):
        wid = lax.axis_index("s") * _NC + lax.axis_index("c")
        base = wid * R
        pltpu.sync_copy(idx_hbm.at[wid], idx_v)
        pltpu.async_copy(table_hbm.at[idx_v.at[0]], rows_v.at[0], gsem)

        def body(c, carry):
            b = lax.rem(c, 2)
            pltpu.make_async_copy(
                table_hbm.at[idx_v.at[c]], rows_v.at[b], gsem
            ).wait()

            @pl.when(c >= 1)
            def _():
                pltpu.make_async_copy(
                    rows_v.at[1 - b],
                    out_hbm.at[pl.ds(base + (c - 1) * C, C)],
                    wsem,
                ).wait()

            @pl.when(c + 1 < K)
            def _():
                pltpu.async_copy(
                    table_hbm.at[idx_v.at[c + 1]], rows_v.at[1 - b], gsem
                )

            pltpu.async_copy(
                rows_v.at[b], out_hbm.at[pl.ds(base + c * C, C)], wsem
            )
            return carry

        lax.fori_loop(0, K, body, 0)
        pltpu.make_async_copy(
            rows_v.at[(K - 1) % 2],
            out_hbm.at[pl.ds(base + (K - 1) * C, C)],
            wsem,
        ).wait()

    return gather_k(table, idx3)


def _interaction_body(a_ref, pn_ref, e_ref,
                      wc_ref, bc_ref, we_ref, be_ref,
                      wg_ref, bg_ref, wm_ref, bm_ref,
                      lns_ref, lnb_ref, o_ref, *, bn, M, D):
    a = a_ref[...]
    phi_c = (
        jnp.dot(a, wc_ref[...], preferred_element_type=jnp.float32)
        + bc_ref[...]
    )
    e = e_ref[...].reshape(bn * M, e_ref.shape[2])
    phi_e = (
        jnp.dot(e, we_ref[...], preferred_element_type=jnp.float32)
        + be_ref[...]
    )
    phi_n = pn_ref[...]
    inter3 = (
        phi_c[:, None, :]
        * phi_n.reshape(bn, M, D)
        * phi_e.reshape(bn, M, D)
    )
    inter = inter3.reshape(bn * M, D).astype(jnp.bfloat16)
    gate = jax.nn.sigmoid(
        jnp.dot(inter, wg_ref[...].astype(jnp.bfloat16),
                preferred_element_type=jnp.float32)
        + bg_ref[...]
    )
    mag = jax.nn.softplus(
        jnp.dot(inter, wm_ref[...].astype(jnp.bfloat16),
                preferred_element_type=jnp.float32)
        + bm_ref[...]
    )
    s = jnp.sum((gate * mag).reshape(bn, M, D), axis=1)
    mean = jnp.mean(s, axis=-1, keepdims=True)
    var = jnp.mean(jnp.square(s - mean), axis=-1, keepdims=True)
    y = (s - mean) * lax.rsqrt(var + 1e-6)
    o_ref[...] = a + y * lns_ref[...] + lnb_ref[...]


def kernel(atom_in_fea, nbr_fea, nbr_fea_idx,
           W_center, b_center, W_nbr, b_nbr, W_edge, b_edge,
           W_gate, b_gate, W_mag, b_mag, ln_scale, ln_bias):
    N, D = atom_in_fea.shape
    M = nbr_fea_idx.shape[1]
    DE = nbr_fea.shape[2]
    B = N * M

    idx_flat = nbr_fea_idx.astype(jnp.int32).reshape(B)

    phi_all = _phi_all(atom_in_fea, W_nbr, b_nbr.reshape(1, D), bn=1000)
    phi_nbr = _sc_gather(phi_all, idx_flat)

    bn = 200
    grid = N // bn
    body = functools.partial(_interaction_body, bn=bn, M=M, D=D)
    out = pl.pallas_call(
        body,
        grid=(grid,),
        in_specs=[
            pl.BlockSpec((bn, D), lambda i: (i, 0)),
            pl.BlockSpec((bn * M, D), lambda i: (i, 0)),
            pl.BlockSpec((bn, M, DE), lambda i: (i, 0, 0)),
            pl.BlockSpec((D, D), lambda i: (0, 0)),
            pl.BlockSpec((1, D), lambda i: (0, 0)),
            pl.BlockSpec((DE, D), lambda i: (0, 0)),
            pl.BlockSpec((1, D), lambda i: (0, 0)),
            pl.BlockSpec((D, D), lambda i: (0, 0)),
            pl.BlockSpec((1, D), lambda i: (0, 0)),
            pl.BlockSpec((D, D), lambda i: (0, 0)),
            pl.BlockSpec((1, D), lambda i: (0, 0)),
            pl.BlockSpec((1, D), lambda i: (0, 0)),
            pl.BlockSpec((1, D), lambda i: (0, 0)),
        ],
        out_specs=pl.BlockSpec((bn, D), lambda i: (i, 0)),
        out_shape=jax.ShapeDtypeStruct((N, D), jnp.float32),
        compiler_params=pltpu.CompilerParams(
            dimension_semantics=("parallel",)
        ),
    )(atom_in_fea, phi_nbr, nbr_fea,
      W_center, b_center.reshape(1, D), W_edge, b_edge.reshape(1, D),
      W_gate, b_gate.reshape(1, D), W_mag, b_mag.reshape(1, D),
      ln_scale.reshape(1, D), ln_bias.reshape(1, D))
    return out


# speedup vs baseline: 1.4492x; 1.4492x over previous
import functools

import jax
import jax.numpy as jnp
from jax import lax
from jax.experimental import pallas as pl
from jax.experimental.pallas import tpu as pltpu
from jax.experimental.pallas import tpu_sc as plsc

_NC = 2
_NS = 16
_NW = _NC * _NS


def _phi_all_body(a_ref, w_ref, b_ref, o_ref):
    o_ref[...] = (
        jnp.dot(a_ref[...], w_ref[...], preferred_element_type=jnp.float32)
        + b_ref[...]
    )


def _phi_all(atom, W, b, bn):
    N, D = atom.shape
    grid = N // bn
    return pl.pallas_call(
        _phi_all_body,
        grid=(grid,),
        in_specs=[
            pl.BlockSpec((bn, D), lambda i: (i, 0)),
            pl.BlockSpec((D, D), lambda i: (0, 0)),
            pl.BlockSpec((1, D), lambda i: (0, 0)),
        ],
        out_specs=pl.BlockSpec((bn, D), lambda i: (i, 0)),
        out_shape=jax.ShapeDtypeStruct((N, D), jnp.float32),
        compiler_params=pltpu.CompilerParams(
            dimension_semantics=("parallel",)
        ),
    )(atom, W, b)


def _sc_gather(table, idx_flat):
    N, D = table.shape
    B = idx_flat.shape[0]
    R = B // _NW
    C = 1
    for cand in (128, 120, 112, 104, 96, 88, 80, 72, 64, 56, 48, 40, 32,
                 24, 16, 8):
        if R % cand == 0:
            C = cand
            break
    K = R // C
    KK = 5 if K % 5 == 0 else 1
    G = K // KK
    idx3 = idx_flat.reshape(_NW, K, C)

    mesh = plsc.VectorSubcoreMesh(
        core_axis_name="c", subcore_axis_name="s",
        num_cores=_NC, num_subcores=_NS,
    )

    @functools.partial(
        pl.kernel,
        mesh=mesh,
        out_type=jax.ShapeDtypeStruct((B, D), jnp.float32),
        scratch_types=[
            pltpu.VMEM((K, C), jnp.int32),
            pltpu.VMEM((2 * KK, C, D), jnp.float32),
            pltpu.SemaphoreType.DMA,
            pltpu.SemaphoreType.DMA,
        ],
        compiler_params=pltpu.CompilerParams(use_tc_tiling_on_sc=True),
    )
    def gather_k(table_hbm, idx_hbm, out_hbm, idx_v, rows_v, gsem, wsem):
        wid = lax.axis_index("s") * _NC + lax.axis_index("c")
        base = wid * R
        pltpu.sync_copy(idx_hbm.at[wid], idx_v)
        for t in range(KK):
            pltpu.async_copy(table_hbm.at[idx_v.at[t]], rows_v.at[t], gsem)

        def body(g, carry):
            bank = lax.rem(g, 2) * KK
            obank = KK - bank
            for t in range(KK):
                pltpu.make_async_copy(
                    table_hbm.at[idx_v.at[g * KK + t]],
                    rows_v.at[bank + t], gsem,
                ).wait()

            @pl.when(g >= 1)
            def _():
                for t in range(KK):
                    pltpu.make_async_copy(
                        rows_v.at[obank + t],
                        out_hbm.at[pl.ds(base + ((g - 1) * KK + t) * C, C)],
                        wsem,
                    ).wait()

            @pl.when(g + 1 < G)
            def _():
                for t in range(KK):
                    pltpu.async_copy(
                        table_hbm.at[idx_v.at[(g + 1) * KK + t]],
                        rows_v.at[obank + t], gsem,
                    )

            for t in range(KK):
                pltpu.async_copy(
                    rows_v.at[bank + t],
                    out_hbm.at[pl.ds(base + (g * KK + t) * C, C)],
                    wsem,
                )
            return carry

        lax.fori_loop(0, G, body, 0)
        fb = ((G - 1) % 2) * KK
        for t in range(KK):
            pltpu.make_async_copy(
                rows_v.at[fb + t],
                out_hbm.at[pl.ds(base + ((G - 1) * KK + t) * C, C)],
                wsem,
            ).wait()

    return gather_k(table, idx3)


def _interaction_body(a_ref, pn_ref, e_ref,
                      wc_ref, bc_ref, we_ref, be_ref,
                      wgm_ref, bgm_ref,
                      lns_ref, lnb_ref, o_ref, *, bn, M, D):
    DE = e_ref.shape[1] // M
    a = a_ref[...]
    phi_c = (
        jnp.dot(a, wc_ref[...], preferred_element_type=jnp.float32)
        + bc_ref[...]
    )
    e2 = e_ref[...]
    we = we_ref[...]
    be = be_ref[...]
    pn3 = pn_ref[...].reshape(M, bn, D)
    inters = []
    for j in range(M):
        ej = e2[:, j * DE:(j + 1) * DE]
        phi_ej = jnp.dot(ej, we, preferred_element_type=jnp.float32) + be
        inters.append(phi_c * pn3[j] * phi_ej)
    inter = jnp.concatenate(inters, axis=0)
    gm = (
        jnp.dot(inter.astype(jnp.bfloat16), wgm_ref[...],
                preferred_element_type=jnp.float32)
        + bgm_ref[...]
    )
    g = gm[:, :D]
    m = gm[:, D:]
    gate = 0.5 * jnp.tanh(0.5 * g) + 0.5
    mag = jnp.maximum(m, 0.0) + jnp.log(1.0 + jnp.exp(-jnp.abs(m)))
    s = jnp.sum((gate * mag).reshape(M, bn, D), axis=0)
    mean = jnp.mean(s, axis=-1, keepdims=True)
    var = jnp.mean(jnp.square(s - mean), axis=-1, keepdims=True)
    y = (s - mean) * lax.rsqrt(var + 1e-6)
    o_ref[...] = a + y * lns_ref[...] + lnb_ref[...]


def kernel(atom_in_fea, nbr_fea, nbr_fea_idx,
           W_center, b_center, W_nbr, b_nbr, W_edge, b_edge,
           W_gate, b_gate, W_mag, b_mag, ln_scale, ln_bias):
    N, D = atom_in_fea.shape
    M = nbr_fea_idx.shape[1]
    DE = nbr_fea.shape[2]
    B = N * M

    S = 5
    Ns = N // S
    bn = 200
    grid = Ns // bn
    idx_seg = (
        nbr_fea_idx.astype(jnp.int32)
        .reshape(S, grid, bn, M)
        .transpose(0, 1, 3, 2)
        .reshape(S, Ns * M)
    )
    nbr2 = nbr_fea.reshape(N, M * DE).astype(jnp.bfloat16)
    W_edge_bf = W_edge.astype(jnp.bfloat16)
    W_gm = jnp.concatenate([W_gate, W_mag], axis=1).astype(jnp.bfloat16)
    b_gm = jnp.concatenate([b_gate, b_mag]).reshape(1, 2 * D)

    phi_all = _phi_all(atom_in_fea, W_nbr, b_nbr.reshape(1, D), bn=1000)
    body = functools.partial(_interaction_body, bn=bn, M=M, D=D)
    outs = []
    pn_segs = [
        _sc_gather(phi_all, idx_seg[s])
        for s in range(S)
    ]
    for s in range(S):
        pn_s = pn_segs[s]
        out_s = pl.pallas_call(
            body,
            grid=(grid,),
            in_specs=[
                pl.BlockSpec((bn, D),
                             lambda i, s=s: (s * grid + i, 0)),
                pl.BlockSpec((bn * M, D), lambda i: (i, 0)),
                pl.BlockSpec((bn, M * DE),
                             lambda i, s=s: (s * grid + i, 0)),
                pl.BlockSpec((D, D), lambda i: (0, 0)),
                pl.BlockSpec((1, D), lambda i: (0, 0)),
                pl.BlockSpec((DE, D), lambda i: (0, 0)),
                pl.BlockSpec((1, D), lambda i: (0, 0)),
                pl.BlockSpec((D, 2 * D), lambda i: (0, 0)),
                pl.BlockSpec((1, 2 * D), lambda i: (0, 0)),
                pl.BlockSpec((1, D), lambda i: (0, 0)),
                pl.BlockSpec((1, D), lambda i: (0, 0)),
            ],
            out_specs=pl.BlockSpec((bn, D), lambda i: (i, 0)),
            out_shape=jax.ShapeDtypeStruct((Ns, D), jnp.float32),
            compiler_params=pltpu.CompilerParams(
                dimension_semantics=("parallel",)
            ),
        )(atom_in_fea, pn_s, nbr2,
          W_center, b_center.reshape(1, D), W_edge_bf, b_edge.reshape(1, D),
          W_gm, b_gm, ln_scale.reshape(1, D), ln_bias.reshape(1, D))
        outs.append(out_s)
    return jnp.concatenate(outs, axis=0)


# speedup vs baseline: 1.4797x; 1.0210x over previous
import functools

import jax
import jax.numpy as jnp
from jax import lax
from jax.experimental import pallas as pl
from jax.experimental.pallas import tpu as pltpu
from jax.experimental.pallas import tpu_sc as plsc

_NC = 2
_NS = 16
_NW = _NC * _NS


def _phi_all_body(a_ref, w_ref, b_ref, o_ref):
    o_ref[...] = (
        jnp.dot(a_ref[...], w_ref[...], preferred_element_type=jnp.float32)
        + b_ref[...]
    )


def _phi_all(atom, W, b, bn):
    N, D = atom.shape
    grid = N // bn
    return pl.pallas_call(
        _phi_all_body,
        grid=(grid,),
        in_specs=[
            pl.BlockSpec((bn, D), lambda i: (i, 0)),
            pl.BlockSpec((D, D), lambda i: (0, 0)),
            pl.BlockSpec((1, D), lambda i: (0, 0)),
        ],
        out_specs=pl.BlockSpec((bn, D), lambda i: (i, 0)),
        out_shape=jax.ShapeDtypeStruct((N, D), jnp.float32),
        compiler_params=pltpu.CompilerParams(
            dimension_semantics=("parallel",)
        ),
    )(atom, W, b)


def _sc_gather(table, idx_flat):
    N, D = table.shape
    B = idx_flat.shape[0]
    R = B // _NW
    C = 1
    for cand in (128, 120, 112, 104, 96, 88, 80, 72, 64, 56, 48, 40, 32,
                 24, 16, 8):
        if R % cand == 0:
            C = cand
            break
    K = R // C
    KK = 5 if K % 5 == 0 else 1
    G = K // KK
    idx3 = idx_flat.reshape(_NW, K, C)

    mesh = plsc.VectorSubcoreMesh(
        core_axis_name="c", subcore_axis_name="s",
        num_cores=_NC, num_subcores=_NS,
    )

    @functools.partial(
        pl.kernel,
        mesh=mesh,
        out_type=jax.ShapeDtypeStruct((B, D), jnp.float32),
        scratch_types=[
            pltpu.VMEM((K, C), jnp.int32),
            pltpu.VMEM((2 * KK, C, D), jnp.float32),
            pltpu.SemaphoreType.DMA,
            pltpu.SemaphoreType.DMA,
        ],
        compiler_params=pltpu.CompilerParams(use_tc_tiling_on_sc=True),
    )
    def gather_k(table_hbm, idx_hbm, out_hbm, idx_v, rows_v, gsem, wsem):
        wid = lax.axis_index("s") * _NC + lax.axis_index("c")
        base = wid * R
        pltpu.sync_copy(idx_hbm.at[wid], idx_v)
        for t in range(KK):
            pltpu.async_copy(table_hbm.at[idx_v.at[t]], rows_v.at[t], gsem)

        def body(g, carry):
            bank = lax.rem(g, 2) * KK
            obank = KK - bank
            for t in range(KK):
                pltpu.make_async_copy(
                    table_hbm.at[idx_v.at[g * KK + t]],
                    rows_v.at[bank + t], gsem,
                ).wait()

            @pl.when(g >= 1)
            def _():
                for t in range(KK):
                    pltpu.make_async_copy(
                        rows_v.at[obank + t],
                        out_hbm.at[pl.ds(base + ((g - 1) * KK + t) * C, C)],
                        wsem,
                    ).wait()

            @pl.when(g + 1 < G)
            def _():
                for t in range(KK):
                    pltpu.async_copy(
                        table_hbm.at[idx_v.at[(g + 1) * KK + t]],
                        rows_v.at[obank + t], gsem,
                    )

            for t in range(KK):
                pltpu.async_copy(
                    rows_v.at[bank + t],
                    out_hbm.at[pl.ds(base + (g * KK + t) * C, C)],
                    wsem,
                )
            return carry

        lax.fori_loop(0, G, body, 0)
        fb = ((G - 1) % 2) * KK
        for t in range(KK):
            pltpu.make_async_copy(
                rows_v.at[fb + t],
                out_hbm.at[pl.ds(base + ((G - 1) * KK + t) * C, C)],
                wsem,
            ).wait()

    return gather_k(table, idx3)


def _interaction_body(a_ref, pn_ref, e_ref,
                      wc_ref, bc_ref, we_ref, be_ref,
                      wgm_ref, bgm_ref,
                      lns_ref, lnb_ref, o_ref, *, bn, M, D):
    DE = e_ref.shape[1] // M
    a = a_ref[...]
    phi_c = (
        jnp.dot(a, wc_ref[...], preferred_element_type=jnp.float32)
        + bc_ref[...]
    )
    e2 = e_ref[...].astype(jnp.bfloat16)
    we = we_ref[...]
    be = be_ref[...]
    pn3 = pn_ref[...].reshape(M, bn, D)
    inters = []
    for j in range(M):
        ej = e2[:, j * DE:(j + 1) * DE]
        phi_ej = jnp.dot(ej, we, preferred_element_type=jnp.float32) + be
        inters.append(phi_c * pn3[j] * phi_ej)
    inter = jnp.concatenate(inters, axis=0)
    gm = (
        jnp.dot(inter.astype(jnp.bfloat16), wgm_ref[...],
                preferred_element_type=jnp.float32)
        + bgm_ref[...]
    )
    g = gm[:, :D]
    m = gm[:, D:]
    gate = 0.5 * jnp.tanh(0.5 * g) + 0.5
    mag = jnp.maximum(m, 0.0) + jnp.log(1.0 + jnp.exp(-jnp.abs(m)))
    s = jnp.sum((gate * mag).reshape(M, bn, D), axis=0)
    mean = jnp.mean(s, axis=-1, keepdims=True)
    var = jnp.mean(jnp.square(s - mean), axis=-1, keepdims=True)
    y = (s - mean) * lax.rsqrt(var + 1e-6)
    o_ref[...] = a + y * lns_ref[...] + lnb_ref[...]


def kernel(atom_in_fea, nbr_fea, nbr_fea_idx,
           W_center, b_center, W_nbr, b_nbr, W_edge, b_edge,
           W_gate, b_gate, W_mag, b_mag, ln_scale, ln_bias):
    N, D = atom_in_fea.shape
    M = nbr_fea_idx.shape[1]
    DE = nbr_fea.shape[2]
    B = N * M

    S = 5
    Ns = N // S
    bn = 200
    grid = Ns // bn
    idx_seg = (
        nbr_fea_idx.astype(jnp.int32)
        .reshape(S, grid, bn, M)
        .transpose(0, 1, 3, 2)
        .reshape(S, Ns * M)
    )
    nbr2 = nbr_fea.reshape(N, M * DE)
    W_edge_bf = W_edge.astype(jnp.bfloat16)
    W_gm = jnp.concatenate([W_gate, W_mag], axis=1).astype(jnp.bfloat16)
    b_gm = jnp.concatenate([b_gate, b_mag]).reshape(1, 2 * D)

    phi_all = _phi_all(atom_in_fea, W_nbr, b_nbr.reshape(1, D), bn=1000)
    body = functools.partial(_interaction_body, bn=bn, M=M, D=D)
    outs = []
    pn_segs = [
        _sc_gather(phi_all, idx_seg[s])
        for s in range(S)
    ]
    for s in range(S):
        pn_s = pn_segs[s]
        out_s = pl.pallas_call(
            body,
            grid=(grid,),
            in_specs=[
                pl.BlockSpec((bn, D),
                             lambda i, s=s: (s * grid + i, 0)),
                pl.BlockSpec((bn * M, D), lambda i: (i, 0)),
                pl.BlockSpec((bn, M * DE),
                             lambda i, s=s: (s * grid + i, 0)),
                pl.BlockSpec((D, D), lambda i: (0, 0)),
                pl.BlockSpec((1, D), lambda i: (0, 0)),
                pl.BlockSpec((DE, D), lambda i: (0, 0)),
                pl.BlockSpec((1, D), lambda i: (0, 0)),
                pl.BlockSpec((D, 2 * D), lambda i: (0, 0)),
                pl.BlockSpec((1, 2 * D), lambda i: (0, 0)),
                pl.BlockSpec((1, D), lambda i: (0, 0)),
                pl.BlockSpec((1, D), lambda i: (0, 0)),
            ],
            out_specs=pl.BlockSpec((bn, D), lambda i: (i, 0)),
            out_shape=jax.ShapeDtypeStruct((Ns, D), jnp.float32),
            compiler_params=pltpu.CompilerParams(
                dimension_semantics=("parallel",)
            ),
        )(atom_in_fea, pn_s, nbr2,
          W_center, b_center.reshape(1, D), W_edge_bf, b_edge.reshape(1, D),
          W_gm, b_gm, ln_scale.reshape(1, D), ln_bias.reshape(1, D))
        outs.append(out_s)
    return jnp.concatenate(outs, axis=0)
